# baseline (speedup 1.0000x reference)
import functools

import jax
import jax.numpy as jnp
from jax import lax
from jax.experimental import pallas as pl
from jax.experimental.pallas import tpu as pltpu
from jax.experimental.pallas import tpu_sc as plsc

_R = 2048
_P = 2_000_000
_D = 8
_L = 16
_NC = 2
_NS = 16
_NW = _NC * _NS
_TILE = 128
_NT = _P // _TILE
_TPW = _NT // _NW
_XTRA = _NT - _TPW * _NW
_CT = 32
_NCHUNK = 15
_CP = _CT * _TILE
_GPC = _CP // _L
_ST = _TPW - _NCHUNK * _CT
_SP = _ST * _TILE
_SGRP = _SP // _L
_TGRP = _TILE // _L


def _tables_body(nodes_ref, wts_ref, tab_ref):
    n = nodes_ref[...]
    w = wts_ref[...]
    for d in range(_D):
        ang = n * ((d + 1) / _D)
        cr = (w * jnp.cos(ang)).astype(jnp.bfloat16)
        ci = (w * jnp.sin(ang)).astype(jnp.bfloat16)
        crw = lax.bitcast_convert_type(cr, jnp.uint16).astype(jnp.uint32)
        ciw = lax.bitcast_convert_type(ci, jnp.uint16).astype(jnp.uint32)
        tab_ref[d] = ((crw << 16) | ciw).astype(jnp.int32)


_tables = pl.pallas_call(
    _tables_body,
    out_shape=jax.ShapeDtypeStruct((_D, 16, 128), jnp.int32),
)


def _cmul(a, b):
    (ar, ai), (br, bi) = a, b
    return (ar * br - ai * bi, ar * bi + ai * br)


def _sc_body(tab_hbm, idx_hbm, out_hbm,
             t0_v, t1_v, t2_v, t3_v, t4_v, t5_v, t6_v, t7_v,
             buf0_v, buf1_v, tail_v, acc_v, sem0, sem1):
    wid = lax.axis_index("s") * _NC + lax.axis_index("c")
    tabs = (t0_v, t1_v, t2_v, t3_v, t4_v, t5_v, t6_v, t7_v)
    for d in range(_D):
        pltpu.sync_copy(tab_hbm.at[pl.ds(d * _R, _R)], tabs[d])

    tile0 = wid * _TPW + jnp.minimum(wid, _XTRA)
    p0 = tile0 * _TILE
    sems = (sem0, sem1)
    bufs = (buf0_v, buf1_v)
    copies = [None, None]
    copies[0] = pltpu.async_copy(idx_hbm.at[:, pl.ds(p0, _CP)], buf0_v, sem0)

    def group_body(bufref, g, acc):
        off = g * _L
        cs = []
        for d in range(_D):
            vals = bufref[d, pl.ds(off, _L)]
            word = plsc.load_gather(tabs[d], [vals])
            c = plsc.bitcast(word & jnp.int32(-65536), jnp.float32)
            s = plsc.bitcast(word << 16, jnp.float32)
            cs.append((c, s))
        while len(cs) > 2:
            cs = [_cmul(cs[i], cs[i + 1]) for i in range(0, len(cs), 2)]
        (ar, ai), (br, bi) = cs
        return acc - (ar * br - ai * bi)

    def process(bufref, ngroups, acc):
        return plsc.parallel_loop(0, ngroups, unroll=4, carry=acc)(
            lambda g, a: group_body(bufref, g, a))

    acc = jnp.zeros((_L,), jnp.float32)
    for ch in range(_NCHUNK):
        if ch + 1 < _NCHUNK:
            copies[(ch + 1) % 2] = pltpu.async_copy(
                idx_hbm.at[:, pl.ds(p0 + (ch + 1) * _CP, _CP)],
                bufs[(ch + 1) % 2], sems[(ch + 1) % 2])
        elif ch + 1 == _NCHUNK:
            copies[(ch + 1) % 2] = pltpu.async_copy(
                idx_hbm.at[:, pl.ds(p0 + _NCHUNK * _CP, _SP)],
                bufs[(ch + 1) % 2].at[:, pl.ds(0, _SP)], sems[(ch + 1) % 2])
        copies[ch % 2].wait()
        acc = process(bufs[ch % 2], _GPC, acc)
    copies[_NCHUNK % 2].wait()
    acc = process(bufs[_NCHUNK % 2], _SGRP, acc)

    tp = jnp.minimum(tile0 + _TPW, _NT - 1) * _TILE
    pltpu.sync_copy(idx_hbm.at[:, pl.ds(tp, _TILE)], tail_v)
    tacc = process(tail_v, _TGRP, jnp.zeros((_L,), jnp.float32))
    acc = acc + jnp.where(wid < _XTRA, tacc, jnp.zeros((_L,), jnp.float32))

    acc_v[...] = acc
    pltpu.sync_copy(acc_v, out_hbm.at[wid])


_sc_compute = functools.partial(
    pl.kernel,
    out_type=jax.ShapeDtypeStruct((_NW, _L), jnp.float32),
    mesh=plsc.VectorSubcoreMesh(core_axis_name="c", subcore_axis_name="s"),
    compiler_params=pltpu.CompilerParams(needs_layout_passes=False),
    scratch_types=(
        [pltpu.VMEM((_R,), jnp.int32) for _ in range(_D)]
        + [
            pltpu.VMEM((_D, _CP), jnp.int32),
            pltpu.VMEM((_D, _CP), jnp.int32),
            pltpu.VMEM((_D, _TILE), jnp.int32),
            pltpu.VMEM((_L,), jnp.float32),
            pltpu.SemaphoreType.DMA,
            pltpu.SemaphoreType.DMA,
        ]
    ),
)(_sc_body)


def kernel(rule_nodes, rule_weights, point_rule_indices):
    idx_t = point_rule_indices.astype(jnp.int32).T
    tab = _tables(rule_nodes.reshape(16, 128), rule_weights.reshape(16, 128))
    parts = _sc_compute(tab.reshape(-1), idx_t)
    return jnp.sum(parts)


# speedup vs baseline: 1.0955x; 1.0955x over previous
import functools

import jax
import jax.numpy as jnp
from jax import lax
from jax.experimental import pallas as pl
from jax.experimental.pallas import tpu as pltpu
from jax.experimental.pallas import tpu_sc as plsc

_R = 2048
_P = 2_000_000
_D = 8
_L = 16
_NC = 2
_NS = 16
_NW = _NC * _NS
_TILE = 128
_NT = _P // _TILE
_TPW = _NT // _NW
_XTRA = _NT - _TPW * _NW
_CT = 32
_NCHUNK = 15
_CP = _CT * _TILE
_GPC = _CP // _L
_ST = _TPW - _NCHUNK * _CT
_SP = _ST * _TILE
_SGRP = _SP // _L
_TGRP = _TILE // _L


def _tables_body(nodes_ref, wts_ref, tab_ref):
    n = nodes_ref[...]
    w = wts_ref[...]
    for d in range(_D):
        ang = n * ((d + 1) / _D)
        cr = (w * jnp.cos(ang)).astype(jnp.bfloat16)
        ci = (w * jnp.sin(ang)).astype(jnp.bfloat16)
        crw = lax.bitcast_convert_type(cr, jnp.uint16).astype(jnp.uint32)
        ciw = lax.bitcast_convert_type(ci, jnp.uint16).astype(jnp.uint32)
        tab_ref[d] = ((crw << 16) | ciw).astype(jnp.int32)


_tables = pl.pallas_call(
    _tables_body,
    out_shape=jax.ShapeDtypeStruct((_D, 16, 128), jnp.int32),
)


def _cmul(a, b):
    (ar, ai), (br, bi) = a, b
    return (ar * br - ai * bi, ar * bi + ai * br)


def _sc_body(tab_hbm, idx_hbm, out_hbm,
             t0_v, t1_v, t2_v, t3_v, t4_v, t5_v, t6_v, t7_v,
             buf0_v, buf1_v, tail_v, acc_v, sem0, sem1):
    wid = lax.axis_index("s") * _NC + lax.axis_index("c")
    tabs = (t0_v, t1_v, t2_v, t3_v, t4_v, t5_v, t6_v, t7_v)
    for d in range(_D):
        pltpu.sync_copy(tab_hbm.at[pl.ds(d * _R, _R)], tabs[d])

    tile0 = wid * _TPW + jnp.minimum(wid, _XTRA)
    p0 = tile0 * _TILE
    sems = (sem0, sem1)
    bufs = (buf0_v, buf1_v)
    copies = [None, None]
    copies[0] = pltpu.async_copy(idx_hbm.at[:, pl.ds(p0, _CP)], buf0_v, sem0)

    def group_body(bufref, g, acc):
        off = g * _L
        cs = []
        for d in range(_D):
            vals = bufref[d, pl.ds(off, _L)]
            word = plsc.load_gather(tabs[d], [vals])
            c = plsc.bitcast(word, jnp.float32)
            s = plsc.bitcast(word << 16, jnp.float32)
            cs.append((c, s))
        while len(cs) > 2:
            cs = [_cmul(cs[i], cs[i + 1]) for i in range(0, len(cs), 2)]
        (ar, ai), (br, bi) = cs
        return acc - (ar * br - ai * bi)

    def process(bufref, ngroups, acc):
        return plsc.parallel_loop(0, ngroups, unroll=2, carry=acc)(
            lambda g, a: group_body(bufref, g, a))

    acc = jnp.zeros((_L,), jnp.float32)
    for ch in range(_NCHUNK):
        if ch + 1 < _NCHUNK:
            copies[(ch + 1) % 2] = pltpu.async_copy(
                idx_hbm.at[:, pl.ds(p0 + (ch + 1) * _CP, _CP)],
                bufs[(ch + 1) % 2], sems[(ch + 1) % 2])
        elif ch + 1 == _NCHUNK:
            copies[(ch + 1) % 2] = pltpu.async_copy(
                idx_hbm.at[:, pl.ds(p0 + _NCHUNK * _CP, _SP)],
                bufs[(ch + 1) % 2].at[:, pl.ds(0, _SP)], sems[(ch + 1) % 2])
        copies[ch % 2].wait()
        acc = process(bufs[ch % 2], _GPC, acc)
    copies[_NCHUNK % 2].wait()
    acc = process(bufs[_NCHUNK % 2], _SGRP, acc)

    tp = jnp.minimum(tile0 + _TPW, _NT - 1) * _TILE
    pltpu.sync_copy(idx_hbm.at[:, pl.ds(tp, _TILE)], tail_v)
    tacc = process(tail_v, _TGRP, jnp.zeros((_L,), jnp.float32))
    acc = acc + jnp.where(wid < _XTRA, tacc, jnp.zeros((_L,), jnp.float32))

    acc_v[...] = acc
    pltpu.sync_copy(acc_v, out_hbm.at[wid])


_sc_compute = functools.partial(
    pl.kernel,
    out_type=jax.ShapeDtypeStruct((_NW, _L), jnp.float32),
    mesh=plsc.VectorSubcoreMesh(core_axis_name="c", subcore_axis_name="s"),
    compiler_params=pltpu.CompilerParams(needs_layout_passes=False),
    scratch_types=(
        [pltpu.VMEM((_R,), jnp.int32) for _ in range(_D)]
        + [
            pltpu.VMEM((_D, _CP), jnp.int32),
            pltpu.VMEM((_D, _CP), jnp.int32),
            pltpu.VMEM((_D, _TILE), jnp.int32),
            pltpu.VMEM((_L,), jnp.float32),
            pltpu.SemaphoreType.DMA,
            pltpu.SemaphoreType.DMA,
        ]
    ),
)(_sc_body)


def kernel(rule_nodes, rule_weights, point_rule_indices):
    idx_t = point_rule_indices.astype(jnp.int32).T
    tab = _tables(rule_nodes.reshape(16, 128), rule_weights.reshape(16, 128))
    parts = _sc_compute(tab.reshape(-1), idx_t)
    return jnp.sum(parts)


# speedup vs baseline: 1.0970x; 1.0014x over previous
import functools

import jax
import jax.numpy as jnp
from jax import lax
from jax.experimental import pallas as pl
from jax.experimental.pallas import tpu as pltpu
from jax.experimental.pallas import tpu_sc as plsc

_R = 2048
_P = 2_000_000
_D = 8
_L = 16
_NC = 2
_NS = 16
_NW = _NC * _NS
_TILE = 128
_NT = _P // _TILE
_TPW = _NT // _NW
_XTRA = _NT - _TPW * _NW
_CT = 32
_NCHUNK = 15
_CP = _CT * _TILE
_GPC = _CP // _L
_ST = _TPW - _NCHUNK * _CT
_SP = _ST * _TILE
_SGRP = _SP // _L
_TGRP = _TILE // _L


def _tables_body(nodes_ref, wts_ref, tab_ref):
    n = nodes_ref[...]
    w = wts_ref[...]
    for d in range(_D):
        ang = n * ((d + 1) / _D)
        crf = w * jnp.cos(ang)
        cif = w * jnp.sin(ang)
        ciw = lax.bitcast_convert_type(
            cif.astype(jnp.bfloat16), jnp.uint16).astype(jnp.uint32)
        h0 = lax.bitcast_convert_type(crf, jnp.uint32) >> 16

        def cand(h):
            wd = (h << 16) | ciw
            return wd, jnp.abs(lax.bitcast_convert_type(wd, jnp.float32) - crf)

        w0, e0 = cand(h0)
        wm, em = cand(h0 - 1)
        wp, ep = cand(h0 + 1)
        best = jnp.where(em < e0, wm, w0)
        beste = jnp.where(em < e0, em, e0)
        best = jnp.where(ep < beste, wp, best)
        tab_ref[d] = lax.bitcast_convert_type(best, jnp.int32)


_tables = pl.pallas_call(
    _tables_body,
    out_shape=jax.ShapeDtypeStruct((_D, 16, 128), jnp.int32),
)


def _cmul(a, b):
    (ar, ai), (br, bi) = a, b
    return (ar * br - ai * bi, ar * bi + ai * br)


def _sc_body(tab_hbm, idx_hbm, out_hbm,
             t0_v, t1_v, t2_v, t3_v, t4_v, t5_v, t6_v, t7_v,
             buf0_v, buf1_v, tail_v, acc_v, sem0, sem1):
    wid = lax.axis_index("s") * _NC + lax.axis_index("c")
    tabs = (t0_v, t1_v, t2_v, t3_v, t4_v, t5_v, t6_v, t7_v)
    for d in range(_D):
        pltpu.sync_copy(tab_hbm.at[pl.ds(d * _R, _R)], tabs[d])

    tile0 = wid * _TPW + jnp.minimum(wid, _XTRA)
    p0 = tile0 * _TILE
    sems = (sem0, sem1)
    bufs = (buf0_v, buf1_v)
    copies = [None, None]
    copies[0] = pltpu.async_copy(idx_hbm.at[:, pl.ds(p0, _CP)], buf0_v, sem0)

    def group_body(bufref, g, acc):
        off = g * _L
        cs = []
        for d in range(_D):
            vals = bufref[d, pl.ds(off, _L)]
            word = plsc.load_gather(tabs[d], [vals])
            c = plsc.bitcast(word, jnp.float32)
            s = plsc.bitcast(word << 16, jnp.float32)
            cs.append((c, s))
        while len(cs) > 2:
            cs = [_cmul(cs[i], cs[i + 1]) for i in range(0, len(cs), 2)]
        (ar, ai), (br, bi) = cs
        return acc - (ar * br - ai * bi)

    def process(bufref, ngroups, acc):
        return plsc.parallel_loop(0, ngroups, unroll=2, carry=acc)(
            lambda g, a: group_body(bufref, g, a))

    acc = jnp.zeros((_L,), jnp.float32)
    for ch in range(_NCHUNK):
        if ch + 1 < _NCHUNK:
            copies[(ch + 1) % 2] = pltpu.async_copy(
                idx_hbm.at[:, pl.ds(p0 + (ch + 1) * _CP, _CP)],
                bufs[(ch + 1) % 2], sems[(ch + 1) % 2])
        elif ch + 1 == _NCHUNK:
            copies[(ch + 1) % 2] = pltpu.async_copy(
                idx_hbm.at[:, pl.ds(p0 + _NCHUNK * _CP, _SP)],
                bufs[(ch + 1) % 2].at[:, pl.ds(0, _SP)], sems[(ch + 1) % 2])
        copies[ch % 2].wait()
        acc = process(bufs[ch % 2], _GPC, acc)
    copies[_NCHUNK % 2].wait()
    acc = process(bufs[_NCHUNK % 2], _SGRP, acc)

    tp = jnp.minimum(tile0 + _TPW, _NT - 1) * _TILE
    pltpu.sync_copy(idx_hbm.at[:, pl.ds(tp, _TILE)], tail_v)
    tacc = process(tail_v, _TGRP, jnp.zeros((_L,), jnp.float32))
    acc = acc + jnp.where(wid < _XTRA, tacc, jnp.zeros((_L,), jnp.float32))

    acc_v[...] = acc
    pltpu.sync_copy(acc_v, out_hbm.at[wid])


_sc_compute = functools.partial(
    pl.kernel,
    out_type=jax.ShapeDtypeStruct((_NW, _L), jnp.float32),
    mesh=plsc.VectorSubcoreMesh(core_axis_name="c", subcore_axis_name="s"),
    compiler_params=pltpu.CompilerParams(needs_layout_passes=False),
    scratch_types=(
        [pltpu.VMEM((_R,), jnp.int32) for _ in range(_D)]
        + [
            pltpu.VMEM((_D, _CP), jnp.int32),
            pltpu.VMEM((_D, _CP), jnp.int32),
            pltpu.VMEM((_D, _TILE), jnp.int32),
            pltpu.VMEM((_L,), jnp.float32),
            pltpu.SemaphoreType.DMA,
            pltpu.SemaphoreType.DMA,
        ]
    ),
)(_sc_body)


def kernel(rule_nodes, rule_weights, point_rule_indices):
    idx_t = point_rule_indices.astype(jnp.int32).T
    tab = _tables(rule_nodes.reshape(16, 128), rule_weights.reshape(16, 128))
    parts = _sc_compute(tab.reshape(-1), idx_t)
    return jnp.sum(parts)


# speedup vs baseline: 1.2068x; 1.1000x over previous
import functools

import jax
import jax.numpy as jnp
from jax import lax
from jax.experimental import pallas as pl
from jax.experimental.pallas import tpu as pltpu
from jax.experimental.pallas import tpu_sc as plsc

_R = 2048
_P = 2_000_000
_D = 8
_L = 16
_NC = 2
_NS = 16
_NW = _NC * _NS
_TILE = 128
_NT = _P // _TILE
_TPW = _NT // _NW
_XTRA = _NT - _TPW * _NW
_CT = 32
_NCHUNK = 15
_CP = _CT * _TILE
_GPC = _CP // _L
_ST = _TPW - _NCHUNK * _CT
_SP = _ST * _TILE
_SGRP = _SP // _L
_TGRP = _TILE // _L


def _tables_body(nodes_ref, wts_ref, tab_ref):
    n = nodes_ref[...]
    w = wts_ref[...]
    for d in range(_D):
        ang = n * ((d + 1) / _D)
        crf = w * jnp.cos(ang)
        cif = w * jnp.sin(ang)
        ciw = lax.bitcast_convert_type(
            cif.astype(jnp.bfloat16), jnp.uint16).astype(jnp.uint32)
        h0 = lax.bitcast_convert_type(crf, jnp.uint32) >> 16

        def cand(h):
            wd = (h << 16) | ciw
            return wd, jnp.abs(lax.bitcast_convert_type(wd, jnp.float32) - crf)

        w0, e0 = cand(h0)
        wm, em = cand(h0 - 1)
        wp, ep = cand(h0 + 1)
        best = jnp.where(em < e0, wm, w0)
        beste = jnp.where(em < e0, em, e0)
        best = jnp.where(ep < beste, wp, best)
        tab_ref[d] = lax.bitcast_convert_type(best, jnp.int32)


_tables = pl.pallas_call(
    _tables_body,
    out_shape=jax.ShapeDtypeStruct((_D, 16, 128), jnp.int32),
)


def _cmul(a, b):
    (ar, ai), (br, bi) = a, b
    return (ar * br - ai * bi, ar * bi + ai * br)


def _sc_body(tab_hbm, idx_hbm, out_hbm,
             t0_v, t1_v, t2_v, t3_v, t4_v, t5_v, t6_v, t7_v,
             buf0_v, buf1_v, tail_v, acc_v, sem0, sem1, sem2):
    wid = lax.axis_index("s") * _NC + lax.axis_index("c")
    tabs = (t0_v, t1_v, t2_v, t3_v, t4_v, t5_v, t6_v, t7_v)
    tile0 = wid * _TPW + jnp.minimum(wid, _XTRA)
    p0 = tile0 * _TILE
    sems = (sem0, sem1)
    bufs = (buf0_v, buf1_v)
    copies = [None, None]

    copies[0] = pltpu.async_copy(idx_hbm.at[:, pl.ds(p0, _CP)], buf0_v, sem0)
    tab_copies = [
        pltpu.async_copy(tab_hbm.at[pl.ds(d * _R, _R)], tabs[d], sem2)
        for d in range(_D)
    ]
    tp = jnp.minimum(tile0 + _TPW, _NT - 1) * _TILE
    tail_copy = pltpu.async_copy(idx_hbm.at[:, pl.ds(tp, _TILE)], tail_v, sem2)
    for c in tab_copies:
        c.wait()

    def group_body(bufref, g, acc):
        off = g * _L
        cs = []
        for d in range(_D):
            vals = bufref[d, pl.ds(off, _L)]
            word = plsc.load_gather(tabs[d], [vals])
            c = plsc.bitcast(word, jnp.float32)
            s = plsc.bitcast(word << 16, jnp.float32)
            cs.append((c, s))
        while len(cs) > 2:
            cs = [_cmul(cs[i], cs[i + 1]) for i in range(0, len(cs), 2)]
        (ar, ai), (br, bi) = cs
        return acc - (ar * br - ai * bi)

    def process(bufref, ngroups, acc):
        return plsc.parallel_loop(0, ngroups, unroll=2, carry=acc)(
            lambda g, a: group_body(bufref, g, a))

    acc = jnp.zeros((_L,), jnp.float32)
    for ch in range(_NCHUNK):
        if ch + 1 < _NCHUNK:
            copies[(ch + 1) % 2] = pltpu.async_copy(
                idx_hbm.at[:, pl.ds(p0 + (ch + 1) * _CP, _CP)],
                bufs[(ch + 1) % 2], sems[(ch + 1) % 2])
        elif ch + 1 == _NCHUNK:
            copies[(ch + 1) % 2] = pltpu.async_copy(
                idx_hbm.at[:, pl.ds(p0 + _NCHUNK * _CP, _SP)],
                bufs[(ch + 1) % 2].at[:, pl.ds(0, _SP)], sems[(ch + 1) % 2])
        copies[ch % 2].wait()
        acc = process(bufs[ch % 2], _GPC, acc)
    copies[_NCHUNK % 2].wait()
    acc = process(bufs[_NCHUNK % 2], _SGRP, acc)

    tail_copy.wait()
    tacc = process(tail_v, _TGRP, jnp.zeros((_L,), jnp.float32))
    acc = acc + jnp.where(wid < _XTRA, tacc, jnp.zeros((_L,), jnp.float32))

    acc_v[...] = acc
    pltpu.sync_copy(acc_v, out_hbm.at[wid])


_sc_compute = functools.partial(
    pl.kernel,
    out_type=jax.ShapeDtypeStruct((_NW, _L), jnp.float32),
    mesh=plsc.VectorSubcoreMesh(core_axis_name="c", subcore_axis_name="s"),
    compiler_params=pltpu.CompilerParams(needs_layout_passes=False),
    scratch_types=(
        [pltpu.VMEM((_R,), jnp.int32) for _ in range(_D)]
        + [
            pltpu.VMEM((_D, _CP), jnp.int32),
            pltpu.VMEM((_D, _CP), jnp.int32),
            pltpu.VMEM((_D, _TILE), jnp.int32),
            pltpu.VMEM((_L,), jnp.float32),
            pltpu.SemaphoreType.DMA,
            pltpu.SemaphoreType.DMA,
            pltpu.SemaphoreType.DMA,
        ]
    ),
)(_sc_body)


def kernel(rule_nodes, rule_weights, point_rule_indices):
    idx_t = point_rule_indices.astype(jnp.int32).T
    tab = _tables(rule_nodes.reshape(16, 128), rule_weights.reshape(16, 128))
    parts = _sc_compute(tab.reshape(-1), idx_t)
    return jnp.sum(parts)
